# baseline (speedup 1.0000x reference)
import functools

import numpy as np
import jax
import jax.numpy as jnp
from jax import lax
from jax.experimental import pallas as pl
from jax.experimental.pallas import tpu as pltpu
from jax.experimental.pallas import tpu_sc as plsc

_VOCAB = 1_000_000
_D = 128
_B = 4096
_H = 200
_NROWS = _B * _H
_NC, _NS = 2, 16
_NW = _NC * _NS
_ROWS_PER_W = _NROWS // _NW
_CHUNK = 128
_NCH = _ROWS_PER_W // _CHUNK
_NBUF = 4
_NSLOT = 2
_SCALE = float(np.sqrt(float(_D)))


def _scale_buf(buf):
    def row(r, carry):
        for c in range(_D // 16):
            sl = (r, pl.ds(c * 16, 16))
            buf[sl] = buf[sl] * _SCALE
        return carry
    lax.fori_loop(0, _CHUNK, row, 0, unroll=2)


@functools.cache
def _build():
    mesh = plsc.VectorSubcoreMesh(
        core_axis_name="c", subcore_axis_name="s",
        num_cores=_NC, num_subcores=_NS)

    @functools.partial(
        pl.kernel,
        out_type=jax.ShapeDtypeStruct((_NROWS, _D), jnp.float32),
        mesh=mesh,
        scratch_types=[
            pltpu.VMEM((_NCH, _CHUNK), jnp.int32),
            *[pltpu.VMEM((_CHUNK, _D), jnp.float32) for _ in range(_NBUF)],
            pltpu.VMEM_SHARED((_NS, _NSLOT, _CHUNK, _D), jnp.float32),
            *[pltpu.SemaphoreType.DMA for _ in range(2 * _NBUF + _NSLOT)],
        ],
    )
    def embed(x_hbm, tab_hbm, out_hbm, idx_v,
              b0, b1, b2, b3, spmem,
              g0, g1, g2, g3, c0, c1, c2, c3, d0, d1):
        bufs = (b0, b1, b2, b3)
        gsems = (g0, g1, g2, g3)
        csems = (c0, c1, c2, c3)
        dsems = (d0, d1)
        cid = lax.axis_index("c")
        sid = lax.axis_index("s")
        wid = sid * _NC + cid
        row0 = wid * _ROWS_PER_W

        pltpu.sync_copy(x_hbm.at[pl.ds(wid * _NCH, _NCH)], idx_v)

        def gather(j, b):
            return pltpu.make_async_copy(
                tab_hbm.at[idx_v.at[j]], bufs[b], gsems[b])

        def crossbar(b, s):
            return pltpu.make_async_copy(bufs[b], spmem.at[sid, s], csems[b])

        def writeback(j, s):
            return pltpu.make_async_copy(
                spmem.at[sid, s],
                out_hbm.at[pl.ds(row0 + j * _CHUNK, _CHUNK)],
                dsems[s])

        gather(0, 0).start()
        gather(1, 1).start()

        def step(g, carry):
            for b in range(_NBUF):
                j = g * _NBUF + b
                f = (b + 2) % _NBUF
                s = b % _NSLOT
                pb = (b + _NBUF - 1) % _NBUF
                jf = j + 2
                if b < 2:
                    gather(jf, f).start()
                else:
                    @pl.when(g <= _NCH // _NBUF - 2)
                    def _():
                        gather(jf, f).start()
                gather(j, b).wait()
                _scale_buf(bufs[b])
                if b >= _NSLOT:
                    writeback(j - _NSLOT, s).wait()
                else:
                    @pl.when(g >= 1)
                    def _():
                        writeback(j - _NSLOT, s).wait()
                crossbar(b, s).start()
                sprev = (b + _NSLOT - 1) % _NSLOT
                if b >= 1:
                    crossbar(pb, sprev).wait()
                    writeback(j - 1, sprev).start()
                else:
                    @pl.when(g >= 1)
                    def _():
                        crossbar(pb, sprev).wait()
                        writeback(j - 1, sprev).start()
            return carry

        lax.fori_loop(0, _NCH // _NBUF, step, 0)

        crossbar((_NCH - 1) % _NBUF, (_NCH - 1) % _NSLOT).wait()
        writeback(_NCH - 2, (_NCH - 2) % _NSLOT).wait()
        writeback(_NCH - 1, (_NCH - 1) % _NSLOT).start()
        writeback(_NCH - 1, (_NCH - 1) % _NSLOT).wait()

    return embed


def kernel(x, emb_weight):
    xf = x.astype(jnp.int32).reshape(_NROWS // _CHUNK, _CHUNK)
    out = _build()(xf, emb_weight)
    return out.reshape(_B, _H, _D)


# speedup vs baseline: 1.0007x; 1.0007x over previous
import functools

import numpy as np
import jax
import jax.numpy as jnp
from jax import lax
from jax.experimental import pallas as pl
from jax.experimental.pallas import tpu as pltpu
from jax.experimental.pallas import tpu_sc as plsc

_VOCAB = 1_000_000
_D = 128
_B = 4096
_H = 200
_NROWS = _B * _H
_NC, _NS = 2, 16
_NW = _NC * _NS
_ROWS_PER_W = _NROWS // _NW
_CHUNK = 128
_NCH = _ROWS_PER_W // _CHUNK
_NBUF = 4
_SCALE = float(np.sqrt(float(_D)))


def _scale_buf(buf):
    def row(r, carry):
        for c in range(_D // 16):
            sl = (r, pl.ds(c * 16, 16))
            buf[sl] = buf[sl] * _SCALE
        return carry
    lax.fori_loop(0, _CHUNK, row, 0, unroll=2)


@functools.cache
def _build():
    mesh = plsc.VectorSubcoreMesh(
        core_axis_name="c", subcore_axis_name="s",
        num_cores=_NC, num_subcores=_NS)

    @functools.partial(
        pl.kernel,
        out_type=jax.ShapeDtypeStruct((_NROWS, _D), jnp.float32),
        mesh=mesh,
        scratch_types=[
            pltpu.VMEM((_NCH, _CHUNK), jnp.int32),
            *[pltpu.VMEM((_CHUNK, _D), jnp.float32) for _ in range(_NBUF)],
            *[pltpu.SemaphoreType.DMA for _ in range(2 * _NBUF)],
        ],
    )
    def embed(x_hbm, tab_hbm, out_hbm, idx_v,
              b0, b1, b2, b3, g0, g1, g2, g3, s0, s1, s2, s3):
        bufs = (b0, b1, b2, b3)
        gsems = (g0, g1, g2, g3)
        ssems = (s0, s1, s2, s3)
        wid = lax.axis_index("s") * _NC + lax.axis_index("c")
        row0 = wid * _ROWS_PER_W

        pltpu.sync_copy(x_hbm.at[pl.ds(wid * _NCH, _NCH)], idx_v)

        def gather(j, b):
            return pltpu.make_async_copy(
                tab_hbm.at[idx_v.at[j]], bufs[b], gsems[b])

        def scatter(j, b):
            return pltpu.make_async_copy(
                bufs[b],
                out_hbm.at[pl.ds(row0 + j * _CHUNK, _CHUNK)],
                ssems[b])

        gather(0, 0).start()
        gather(1, 1).start()

        def step(g, carry):
            for b in range(_NBUF):
                j = g * _NBUF + b
                f = (b + 2) % _NBUF
                jf = j + 2
                if b < 2:
                    @pl.when(g >= 1)
                    def _():
                        scatter(jf - _NBUF, f).wait()
                    gather(jf, f).start()
                else:
                    @pl.when(g <= _NCH // _NBUF - 2)
                    def _():
                        scatter(jf - _NBUF, f).wait()
                        gather(jf, f).start()
                gather(j, b).wait()
                _scale_buf(bufs[b])
                scatter(j, b).start()
            return carry

        lax.fori_loop(0, _NCH // _NBUF, step, 0)

        for b in range(_NBUF):
            scatter(_NCH - _NBUF + b, b).wait()

    return embed


def kernel(x, emb_weight):
    xf = x.astype(jnp.int32).reshape(_NROWS // _CHUNK, _CHUNK)
    out = _build()(xf, emb_weight)
    return out.reshape(_B, _H, _D)


# speedup vs baseline: 1.0015x; 1.0009x over previous
import functools

import numpy as np
import jax
import jax.numpy as jnp
from jax import lax
from jax.experimental import pallas as pl
from jax.experimental.pallas import tpu as pltpu
from jax.experimental.pallas import tpu_sc as plsc

_VOCAB = 1_000_000
_D = 128
_B = 4096
_H = 200
_NROWS = _B * _H
_NC, _NS = 2, 16
_NW = _NC * _NS
_ROWS_PER_W = _NROWS // _NW
_CHUNK = 128
_NCH = _ROWS_PER_W // _CHUNK
_NBUF = 5
_SCALE = float(np.sqrt(float(_D)))


def _scale_buf(buf):
    def row(r, carry):
        for c in range(_D // 16):
            sl = (r, pl.ds(c * 16, 16))
            buf[sl] = buf[sl] * _SCALE
        return carry
    lax.fori_loop(0, _CHUNK, row, 0, unroll=2)


@functools.cache
def _build():
    mesh = plsc.VectorSubcoreMesh(
        core_axis_name="c", subcore_axis_name="s",
        num_cores=_NC, num_subcores=_NS)

    @functools.partial(
        pl.kernel,
        out_type=jax.ShapeDtypeStruct((_NROWS, _D), jnp.float32),
        mesh=mesh,
        scratch_types=[
            pltpu.VMEM((_NCH, _CHUNK), jnp.int32),
            *[pltpu.VMEM((_CHUNK, _D), jnp.float32) for _ in range(_NBUF)],
            *[pltpu.SemaphoreType.DMA for _ in range(2 * _NBUF)],
        ],
    )
    def embed(x_hbm, tab_hbm, out_hbm, idx_v, *scr):
        bufs = scr[:_NBUF]
        gsems = scr[_NBUF:2 * _NBUF]
        ssems = scr[2 * _NBUF:]
        wid = lax.axis_index("s") * _NC + lax.axis_index("c")
        row0 = wid * _ROWS_PER_W

        pltpu.sync_copy(x_hbm.at[pl.ds(wid * _NCH, _NCH)], idx_v)

        def gather(j, b):
            return pltpu.make_async_copy(
                tab_hbm.at[idx_v.at[j]], bufs[b], gsems[b])

        def scatter(j, b):
            return pltpu.make_async_copy(
                bufs[b],
                out_hbm.at[pl.ds(row0 + j * _CHUNK, _CHUNK)],
                ssems[b])

        gather(0, 0).start()
        gather(1, 1).start()

        def step(g, carry):
            for b in range(_NBUF):
                j = g * _NBUF + b
                f = (b + 2) % _NBUF
                jf = j + 2
                if b < _NBUF - 2:
                    @pl.when(g >= 1)
                    def _():
                        scatter(jf - _NBUF, f).wait()
                    gather(jf, f).start()
                else:
                    @pl.when(g <= _NCH // _NBUF - 2)
                    def _():
                        scatter(jf - _NBUF, f).wait()
                        gather(jf, f).start()
                gather(j, b).wait()
                _scale_buf(bufs[b])
                scatter(j, b).start()
            return carry

        lax.fori_loop(0, _NCH // _NBUF, step, 0)

        for b in range(_NBUF):
            scatter(_NCH - _NBUF + b, b).wait()

    return embed


def kernel(x, emb_weight):
    xf = x.astype(jnp.int32).reshape(_NROWS // _CHUNK, _CHUNK)
    out = _build()(xf, emb_weight)
    return out.reshape(_B, _H, _D)


# speedup vs baseline: 1.0034x; 1.0018x over previous
import functools

import numpy as np
import jax
import jax.numpy as jnp
from jax import lax
from jax.experimental import pallas as pl
from jax.experimental.pallas import tpu as pltpu
from jax.experimental.pallas import tpu_sc as plsc

_VOCAB = 1_000_000
_D = 128
_B = 4096
_H = 200
_NROWS = _B * _H
_NC, _NS = 2, 16
_NW = _NC * _NS
_ROWS_PER_W = _NROWS // _NW
_CHUNK = 128
_NCH = _ROWS_PER_W // _CHUNK
_NBUF = 5
_LOOK = 3
_SCALE = float(np.sqrt(float(_D)))


def _scale_buf(buf):
    def row(r, carry):
        for c in range(_D // 16):
            sl = (r, pl.ds(c * 16, 16))
            buf[sl] = buf[sl] * _SCALE
        return carry
    lax.fori_loop(0, _CHUNK, row, 0, unroll=2)


@functools.cache
def _build():
    mesh = plsc.VectorSubcoreMesh(
        core_axis_name="c", subcore_axis_name="s",
        num_cores=_NC, num_subcores=_NS)

    @functools.partial(
        pl.kernel,
        out_type=jax.ShapeDtypeStruct((_NROWS, _D), jnp.float32),
        mesh=mesh,
        scratch_types=[
            pltpu.VMEM((_NCH, _CHUNK), jnp.int32),
            *[pltpu.VMEM((_CHUNK, _D), jnp.float32) for _ in range(_NBUF)],
            *[pltpu.SemaphoreType.DMA for _ in range(2 * _NBUF)],
        ],
    )
    def embed(x_hbm, tab_hbm, out_hbm, idx_v, *scr):
        bufs = scr[:_NBUF]
        gsems = scr[_NBUF:2 * _NBUF]
        ssems = scr[2 * _NBUF:]
        wid = lax.axis_index("s") * _NC + lax.axis_index("c")
        row0 = wid * _ROWS_PER_W

        pltpu.sync_copy(x_hbm.at[pl.ds(wid * _NCH, _NCH)], idx_v)

        def gather(j, b):
            return pltpu.make_async_copy(
                tab_hbm.at[idx_v.at[j]], bufs[b], gsems[b])

        def scatter(j, b):
            return pltpu.make_async_copy(
                bufs[b],
                out_hbm.at[pl.ds(row0 + j * _CHUNK, _CHUNK)],
                ssems[b])

        for b in range(_LOOK):
            gather(b, b).start()

        def step(g, carry):
            for b in range(_NBUF):
                j = g * _NBUF + b
                f = (b + _LOOK) % _NBUF
                jf = j + _LOOK
                if b < _NBUF - _LOOK:
                    @pl.when(g >= 1)
                    def _():
                        scatter(jf - _NBUF, f).wait()
                    gather(jf, f).start()
                else:
                    @pl.when(g <= _NCH // _NBUF - 2)
                    def _():
                        scatter(jf - _NBUF, f).wait()
                        gather(jf, f).start()
                gather(j, b).wait()
                _scale_buf(bufs[b])
                scatter(j, b).start()
            return carry

        lax.fori_loop(0, _NCH // _NBUF, step, 0)

        for b in range(_NBUF):
            scatter(_NCH - _NBUF + b, b).wait()

    return embed


def kernel(x, emb_weight):
    xf = x.astype(jnp.int32).reshape(_NROWS // _CHUNK, _CHUNK)
    out = _build()(xf, emb_weight)
    return out.reshape(_B, _H, _D)
